# baseline (speedup 1.0000x reference)
import jax
import jax.numpy as jnp
from jax.experimental import pallas as pl
from jax.experimental.pallas import tpu as pltpu

_PARTS = [[0, 1, 2, 3, 20], [4, 5, 6, 7, 21, 22], [8, 9, 10, 11, 23, 24],
          [12, 13, 14, 15], [16, 17, 18, 19]]
_V_OUT = 25


_NSPLIT = 4


def _body(x_hbm, o_hbm, vbuf, in_sems, out_sem):
    n = x_hbm.shape[0]
    bn = n // _NSPLIT
    in_cps = []
    for i in range(5):
        for j in range(_NSPLIT):
            ns = pl.ds(j * bn, bn)
            cp = pltpu.make_async_copy(x_hbm.at[ns, i], vbuf.at[i, ns],
                                       in_sems.at[i])
            cp.start()
            in_cps.append(cp)
    out_cps = []
    for pi, part in enumerate(_PARTS):
        for cp in in_cps[pi * _NSPLIT:(pi + 1) * _NSPLIT]:
            cp.wait()
        for v in part:
            for j in range(_NSPLIT):
                ns = pl.ds(j * bn, bn)
                cp = pltpu.make_async_copy(vbuf.at[pi, ns], o_hbm.at[ns, v],
                                           out_sem)
                cp.start()
                out_cps.append(cp)
    for cp in out_cps:
        cp.wait()


def kernel(x):
    N, C, T, V = x.shape
    xt = jnp.transpose(x, (0, 3, 1, 2))
    out_t = pl.pallas_call(
        _body,
        in_specs=[pl.BlockSpec(memory_space=pl.ANY)],
        out_specs=pl.BlockSpec(memory_space=pl.ANY),
        out_shape=jax.ShapeDtypeStruct((N, _V_OUT, C, T), x.dtype),
        scratch_shapes=[
            pltpu.VMEM((V, N, C, T), jnp.float32),
            pltpu.SemaphoreType.DMA((V,)),
            pltpu.SemaphoreType.DMA,
        ],
    )(xt)
    return jnp.transpose(out_t, (0, 2, 3, 1))
